# Initial kernel scaffold; baseline (speedup 1.0000x reference)
import jax
import jax.numpy as jnp
from jax.experimental import pallas as pl


def kernel(x, meta_W, meta_b, macro_W, macro_b, fc1_W, fc1_b, fc2_W, fc2_b, fc3_W, fc3_b):
    raise NotImplementedError("write your pallas kernel here")


# speedup vs baseline: 2.1833x; 2.1833x over previous
import functools

import jax
import jax.numpy as jnp
from jax import lax
from jax.experimental import pallas as pl
from jax.experimental.pallas import tpu as pltpu
from jax.experimental.pallas import tpu_sc as plsc

D = 1024
H = 1024
O = 1024
N = 2048
MG = 2
MAC = 3
MIC = 4
G = MG * MAC
ALPHA = 0.01

B = 256
KMAX = N // B
NBMAX = N // B + G
NPAD = N + G * B

_NC, _NS = 2, 16
_NW = _NC * _NS
_SLOTS_W = NPAD // _NW
_TOKS_W = N // _NW


def _router_body(x_ref, w_ref, b_ref, ids_ref, aux_ref):
    x = x_ref[...]
    logits = jnp.dot(x, w_ref[...], preferred_element_type=jnp.float32)
    logits = logits + b_ref[...]
    nf = jnp.float32(N)

    a0 = logits[:, 0:1]
    a1 = logits[:, 1:2]
    mx = jnp.maximum(a0, a1)
    e0 = jnp.exp(a0 - mx)
    e1 = jnp.exp(a1 - mx)
    s = e0 + e1
    f0 = jnp.sum(e0 / s) / nf
    f1 = jnp.sum(e1 / s) / nf
    aux = ALPHA * 2.0 * (f0 * f0 + f1 * f1)

    topi = (a1 > a0).astype(jnp.int32)
    msel = [None, None]
    for g in range(MG):
        base = MG + MAC * g
        c0 = logits[:, base:base + 1]
        c1 = logits[:, base + 1:base + 2]
        c2 = logits[:, base + 2:base + 3]
        m = jnp.maximum(jnp.maximum(c0, c1), c2)
        x0 = jnp.exp(c0 - m)
        x1 = jnp.exp(c1 - m)
        x2 = jnp.exp(c2 - m)
        ssum = x0 + x1 + x2
        maskg = (topi == g).astype(jnp.float32)
        cnt = jnp.sum(maskg)
        denom = jnp.maximum(cnt, 1.0)
        fj0 = jnp.sum(x0 / ssum * maskg) / denom
        fj1 = jnp.sum(x1 / ssum * maskg) / denom
        fj2 = jnp.sum(x2 / ssum * maskg) / denom
        lb = ALPHA * 3.0 * (fj0 * fj0 + fj1 * fj1 + fj2 * fj2)
        aux = aux + jnp.where(cnt > 0.0, lb, 0.0)
        msel[g] = jnp.where(c1 > c0,
                            jnp.where(c2 > c1, 2, 1),
                            jnp.where(c2 > c0, 2, 0)).astype(jnp.int32)

    ids = topi * MAC + jnp.where(topi == 1, msel[1], msel[0])
    ids_ref[...] = ids
    aux_ref[...] = jnp.full((8, 128), aux, jnp.float32)


def _router(x, wcat, bcat):
    return pl.pallas_call(
        _router_body,
        out_shape=[
            jax.ShapeDtypeStruct((N, 1), jnp.int32),
            jax.ShapeDtypeStruct((8, 128), jnp.float32),
        ],
    )(x, wcat, bcat)


@functools.lru_cache(maxsize=1)
def _sc_gathers():
    mesh = plsc.VectorSubcoreMesh(core_axis_name="c", subcore_axis_name="s",
                                  num_cores=_NC, num_subcores=_NS)

    @functools.partial(
        pl.kernel,
        out_type=jax.ShapeDtypeStruct((NPAD, D), jnp.float32),
        mesh=mesh,
        scratch_types=[
            pltpu.VMEM((_SLOTS_W,), jnp.int32),
            pltpu.VMEM((_SLOTS_W, D), jnp.float32),
            pltpu.SemaphoreType.DMA,
        ],
    )
    def gather_in(x_hbm, idx_hbm, out_hbm, idx_v, rows_v, sem):
        wid = lax.axis_index("s") * _NC + lax.axis_index("c")
        base = wid * _SLOTS_W
        pltpu.sync_copy(idx_hbm.at[pl.ds(base, _SLOTS_W)], idx_v)
        pltpu.async_copy(x_hbm.at[idx_v], rows_v, sem).wait()
        pltpu.sync_copy(rows_v, out_hbm.at[pl.ds(base, _SLOTS_W)])

    @functools.partial(
        pl.kernel,
        out_type=jax.ShapeDtypeStruct((N, O), jnp.float32),
        mesh=mesh,
        scratch_types=[
            pltpu.VMEM((_TOKS_W,), jnp.int32),
            pltpu.VMEM((_TOKS_W, O), jnp.float32),
            pltpu.SemaphoreType.DMA,
        ],
    )
    def gather_out(tab_hbm, idx_hbm, out_hbm, idx_v, rows_v, sem):
        wid = lax.axis_index("s") * _NC + lax.axis_index("c")
        base = wid * _TOKS_W
        pltpu.sync_copy(idx_hbm.at[pl.ds(base, _TOKS_W)], idx_v)
        pltpu.async_copy(tab_hbm.at[idx_v], rows_v, sem).wait()
        pltpu.sync_copy(rows_v, out_hbm.at[pl.ds(base, _TOKS_W)])

    return gather_in, gather_out


def _ffn_body(g_ref, r_ref, v_ref, xs_ref, w1_ref, b1_ref, w2_ref, b2_ref,
              w3_ref, b3_ref, out_ref):
    i = pl.program_id(0)
    e = pl.program_id(1)
    valid = v_ref[i] == 1

    @pl.when(valid)
    def _():
        xb = xs_ref[...]
        h = jnp.dot(xb, w1_ref[0, 0], preferred_element_type=jnp.float32)
        h = jnp.maximum(h + b1_ref[0, 0], 0.0)
        h2 = jnp.dot(h, w2_ref[0, 0], preferred_element_type=jnp.float32)
        h2 = jnp.maximum(h2 + b2_ref[0, 0] + xb, 0.0)
        oe = jnp.dot(h2, w3_ref[0, 0], preferred_element_type=jnp.float32)
        oe = (oe + b3_ref[0, 0]) * (1.0 / MIC)

        @pl.when(e == 0)
        def _():
            out_ref[...] = oe

        @pl.when(e > 0)
        def _():
            out_ref[...] += oe


def _ffn(blk_gid, blk_row, blk_val, xs_pad, fc1_W, b1r, fc2_W, b2r, fc3_W, b3r):
    def _e_eff(e, v, i):
        return jnp.where(v[i] == 1, e, MIC - 1)

    grid_spec = pltpu.PrefetchScalarGridSpec(
        num_scalar_prefetch=3,
        grid=(NBMAX, MIC),
        in_specs=[
            pl.BlockSpec((B, D), lambda i, e, g, r, v: (r[i], 0)),
            pl.BlockSpec((1, 1, D, H),
                         lambda i, e, g, r, v: (g[i], _e_eff(e, v, i), 0, 0)),
            pl.BlockSpec((1, 1, H),
                         lambda i, e, g, r, v: (g[i] * MIC + _e_eff(e, v, i), 0, 0)),
            pl.BlockSpec((1, 1, H, H),
                         lambda i, e, g, r, v: (g[i], _e_eff(e, v, i), 0, 0)),
            pl.BlockSpec((1, 1, H),
                         lambda i, e, g, r, v: (g[i] * MIC + _e_eff(e, v, i), 0, 0)),
            pl.BlockSpec((1, 1, H, O),
                         lambda i, e, g, r, v: (g[i], _e_eff(e, v, i), 0, 0)),
            pl.BlockSpec((1, 1, O),
                         lambda i, e, g, r, v: (g[i] * MIC + _e_eff(e, v, i), 0, 0)),
        ],
        out_specs=pl.BlockSpec((B, O), lambda i, e, g, r, v: (r[i], 0)),
    )
    return pl.pallas_call(
        _ffn_body,
        grid_spec=grid_spec,
        out_shape=jax.ShapeDtypeStruct((NPAD, O), jnp.float32),
        compiler_params=pltpu.CompilerParams(
            dimension_semantics=("arbitrary", "arbitrary")),
    )(blk_gid, blk_row, blk_val, xs_pad, fc1_W, b1r, fc2_W, b2r, fc3_W, b3r)


def kernel(x, meta_W, meta_b, macro_W, macro_b,
           fc1_W, fc1_b, fc2_W, fc2_b, fc3_W, fc3_b):
    wcat = jnp.concatenate(
        [meta_W, macro_W[0], macro_W[1],
         jnp.zeros((D, 128 - MG - MG * MAC), jnp.float32)], axis=1)
    bcat = jnp.concatenate(
        [meta_b, macro_b[0], macro_b[1],
         jnp.zeros((128 - MG - MG * MAC,), jnp.float32)])[None, :]

    ids2d, aux2d = _router(x, wcat, bcat)
    ids = ids2d[:, 0]
    aux = aux2d[0, 0]

    c6 = jnp.arange(G, dtype=jnp.int32)
    oh = (ids[:, None] == c6[None, :]).astype(jnp.int32)
    counts = jnp.sum(oh, axis=0)
    ranks = jnp.take_along_axis(jnp.cumsum(oh, axis=0) - 1,
                                ids[:, None], axis=1)[:, 0]
    nb = (counts + B - 1) // B
    astart = jnp.concatenate(
        [jnp.zeros((1,), jnp.int32), jnp.cumsum(nb * B)])[:G]
    p_tok = astart[ids] + ranks
    tok_for_slot = jnp.zeros((NPAD,), jnp.int32).at[p_tok].set(
        jnp.arange(N, dtype=jnp.int32))

    total_nb = jnp.sum(nb)
    cand_gid = jnp.repeat(c6, KMAX)
    cand_k = jnp.tile(jnp.arange(KMAX, dtype=jnp.int32), G)
    cand_valid = cand_k < nb[cand_gid]
    cand_row = astart[cand_gid] // B + cand_k
    order = jnp.argsort(jnp.logical_not(cand_valid), stable=True)
    g_s = cand_gid[order][:NBMAX]
    r_s = cand_row[order][:NBMAX]
    j = jnp.arange(NBMAX, dtype=jnp.int32)
    g_last = g_s[total_nb - 1]
    r_last = r_s[total_nb - 1]
    blk_gid = jnp.where(j < total_nb, g_s, g_last).astype(jnp.int32)
    blk_row = jnp.where(j < total_nb, r_s, r_last).astype(jnp.int32)
    blk_val = (j < total_nb).astype(jnp.int32)

    gather_in, gather_out = _sc_gathers()
    xs_pad = gather_in(x, tok_for_slot)

    b1r = fc1_b.reshape(G * MIC, 1, H)
    b2r = fc2_b.reshape(G * MIC, 1, H)
    b3r = fc3_b.reshape(G * MIC, 1, O)
    out_pad = _ffn(blk_gid, blk_row, blk_val, xs_pad,
                   fc1_W, b1r, fc2_W, b2r, fc3_W, b3r)

    final = gather_out(out_pad, p_tok)
    return final, aux
